# Initial kernel scaffold; baseline (speedup 1.0000x reference)
import jax
import jax.numpy as jnp
from jax.experimental import pallas as pl


def kernel(x, edge_index, W, b, gamma, beta):
    raise NotImplementedError("write your pallas kernel here")


# speedup vs baseline: 20.4060x; 20.4060x over previous
import functools

import jax
import jax.numpy as jnp
from jax import lax
from jax.experimental import pallas as pl
from jax.experimental.pallas import tpu as pltpu
from jax.experimental.pallas import tpu_sc as plsc

N = 10000
NPAD = 10240
D = 128
NC = 2
NS = 16
NW = NC * NS
RPT = NPAD // NS
CHUNK = 80


def _mesh():
    return plsc.VectorSubcoreMesh(core_axis_name="c", subcore_axis_name="s")


def _hist_call(dst3, ones_pat, zrows):
    nchunk = dst3.shape[1]

    @functools.partial(
        pl.kernel,
        out_type=jax.ShapeDtypeStruct((NC, NPAD, D), jnp.float32),
        mesh=_mesh(),
        scratch_types=[
            pltpu.VMEM((nchunk, CHUNK), jnp.int32),
            pltpu.VMEM((CHUNK, D), jnp.float32),
            pltpu.VMEM_SHARED((NPAD, D), jnp.float32),
        ],
    )
    def hist(dst_hbm, ones_hbm, zeros_hbm, out_hbm, didx, ones_v, deg_sp):
        c = lax.axis_index("c")
        s = lax.axis_index("s")
        wid = c * NS + s
        pltpu.sync_copy(dst_hbm.at[wid], didx)
        pltpu.sync_copy(ones_hbm, ones_v)
        pltpu.sync_copy(zeros_hbm, deg_sp.at[pl.ds(s * RPT, RPT)])
        plsc.subcore_barrier()

        def body(j, carry):
            pltpu.sync_copy(ones_v, deg_sp.at[didx.at[j]], add=True)
            return carry

        lax.fori_loop(0, nchunk, body, 0)
        plsc.subcore_barrier()
        pltpu.sync_copy(deg_sp.at[pl.ds(s * RPT, RPT)],
                        out_hbm.at[c, pl.ds(s * RPT, RPT)])

    return hist(dst3, ones_pat, zrows)


def _prep_call(x, W, degp):
    def prep(x_ref, w_ref, degp_ref, g_ref, dinv_ref):
        deg = degp_ref[0, :N, 0:1] + degp_ref[1, :N, 0:1] + 1.0
        dinv = lax.rsqrt(deg)
        h = jnp.dot(x_ref[...], w_ref[...], preferred_element_type=jnp.float32)
        g_ref[...] = h * dinv
        dinv_ref[...] = dinv

    return pl.pallas_call(
        prep,
        out_shape=(jax.ShapeDtypeStruct((N, D), jnp.float32),
                   jax.ShapeDtypeStruct((N, 1), jnp.float32)),
    )(x, W, degp)


def _aggregate_call(g, src3, dst3, zrows):
    nchunk = src3.shape[1]

    @functools.partial(
        pl.kernel,
        out_type=jax.ShapeDtypeStruct((NC, NPAD, D), jnp.float32),
        mesh=_mesh(),
        scratch_types=[
            pltpu.VMEM((nchunk, CHUNK), jnp.int32),
            pltpu.VMEM((nchunk, CHUNK), jnp.int32),
            pltpu.VMEM((CHUNK, D), jnp.float32),
            pltpu.VMEM_SHARED((NPAD, D), jnp.float32),
            pltpu.SemaphoreType.DMA,
        ],
    )
    def agg_k(g_hbm, src_hbm, dst_hbm, z_hbm, out_hbm,
              sidx, didx, rows, agg_sp, sem):
        c = lax.axis_index("c")
        s = lax.axis_index("s")
        wid = c * NS + s
        pltpu.sync_copy(src_hbm.at[wid], sidx)
        pltpu.sync_copy(dst_hbm.at[wid], didx)
        pltpu.sync_copy(z_hbm, agg_sp.at[pl.ds(s * RPT, RPT)])
        plsc.subcore_barrier()

        def body(j, carry):
            pltpu.async_copy(g_hbm.at[sidx.at[j]], rows, sem).wait()
            pltpu.sync_copy(rows, agg_sp.at[didx.at[j]], add=True)
            return carry

        lax.fori_loop(0, nchunk, body, 0)
        plsc.subcore_barrier()
        pltpu.sync_copy(agg_sp.at[pl.ds(s * RPT, RPT)],
                        out_hbm.at[c, pl.ds(s * RPT, RPT)])

    return agg_k(g, src3, dst3, zrows)


def _finish_call(S, g, dinv, x, b, gamma, beta):
    def fin(s_ref, g_ref, dinv_ref, x_ref, b_ref, gm_ref, bt_ref, o_ref):
        agg = (s_ref[0, :N] + s_ref[1, :N] + g_ref[...]) * dinv_ref[...] + b_ref[...]
        mean = jnp.mean(agg, axis=0, keepdims=True)
        ctr = agg - mean
        var = jnp.mean(ctr * ctr, axis=0, keepdims=True)
        xhat = ctr * lax.rsqrt(var + 1e-5)
        o_ref[...] = jnp.maximum(gm_ref[...] * xhat + bt_ref[...], 0.0) + x_ref[...]

    return pl.pallas_call(
        fin,
        out_shape=jax.ShapeDtypeStruct((N, D), jnp.float32),
    )(S, g, dinv, x, b.reshape(1, D), gamma.reshape(1, D), beta.reshape(1, D))


def kernel(x, edge_index, W, b, gamma, beta):
    E = edge_index.shape[1]
    per_w = E // NW
    nchunk = per_w // CHUNK
    assert per_w * NW == E and nchunk * CHUNK == per_w
    ei = edge_index.astype(jnp.int32)
    src3 = ei[0].reshape(NW, nchunk, CHUNK)
    dst3 = ei[1].reshape(NW, nchunk, CHUNK)
    ones_pat = jnp.zeros((CHUNK, D), jnp.float32).at[:, 0].set(1.0)
    zrows = jnp.zeros((RPT, D), jnp.float32)

    degp = _hist_call(dst3, ones_pat, zrows)
    g, dinv = _prep_call(x, W, degp)
    S = _aggregate_call(g, src3, dst3, zrows)
    return _finish_call(S, g, dinv, x, b, gamma, beta)


# speedup vs baseline: 27.3448x; 1.3400x over previous
import functools

import jax
import jax.numpy as jnp
from jax import lax
from jax.experimental import pallas as pl
from jax.experimental.pallas import tpu as pltpu
from jax.experimental.pallas import tpu_sc as plsc

N = 10000
NPAD = 10240
D = 128
NC = 2
NS = 16
NW = NC * NS
RPT = NPAD // NS
CHUNK = 125
IB = 16


def _mesh():
    return plsc.VectorSubcoreMesh(core_axis_name="c", subcore_axis_name="s")


def _hist_call(dst3, ones_pat, zrows):
    nchunk = dst3.shape[1]

    @functools.partial(
        pl.kernel,
        out_type=jax.ShapeDtypeStruct((NC, NPAD, D), jnp.float32),
        mesh=_mesh(),
        scratch_types=[
            pltpu.VMEM((nchunk, CHUNK), jnp.int32),
            pltpu.VMEM((CHUNK, D), jnp.float32),
            pltpu.VMEM_SHARED((NPAD, D), jnp.float32),
        ],
    )
    def hist(dst_hbm, ones_hbm, zeros_hbm, out_hbm, didx, ones_v, deg_sp):
        c = lax.axis_index("c")
        s = lax.axis_index("s")
        wid = c * NS + s
        pltpu.sync_copy(dst_hbm.at[wid], didx)
        pltpu.sync_copy(ones_hbm, ones_v)
        pltpu.sync_copy(zeros_hbm, deg_sp.at[pl.ds(s * RPT, RPT)])
        plsc.subcore_barrier()

        def body(j, carry):
            pltpu.sync_copy(ones_v, deg_sp.at[didx.at[j]], add=True)
            return carry

        lax.fori_loop(0, nchunk, body, 0)
        plsc.subcore_barrier()
        pltpu.sync_copy(deg_sp.at[pl.ds(s * RPT, RPT)],
                        out_hbm.at[c, pl.ds(s * RPT, RPT)])

    return hist(dst3, ones_pat, zrows)


def _prep_call(x, W, degp):
    def prep(x_ref, w_ref, degp_ref, g_ref, dinv_ref):
        deg = degp_ref[0, :N, 0:1] + degp_ref[1, :N, 0:1] + 1.0
        dinv = lax.rsqrt(deg)
        h = jnp.dot(x_ref[...], w_ref[...], preferred_element_type=jnp.float32)
        g_ref[...] = h * dinv
        dinv_ref[...] = dinv

    return pl.pallas_call(
        prep,
        out_shape=(jax.ShapeDtypeStruct((N, D), jnp.float32),
                   jax.ShapeDtypeStruct((N, 1), jnp.float32)),
    )(x, W, degp)


def _aggregate_call(g, src3, dst3, zrows):
    nchunk = src3.shape[1]
    nblk = nchunk // IB
    assert nblk * IB == nchunk and IB % 2 == 0 and IB % 8 == 0

    @functools.partial(
        pl.kernel,
        out_type=jax.ShapeDtypeStruct((NC, NPAD, D), jnp.float32),
        mesh=_mesh(),
        scratch_types=[
            pltpu.VMEM((IB, CHUNK), jnp.int32),
            pltpu.VMEM((IB, CHUNK), jnp.int32),
            pltpu.VMEM((CHUNK, D), jnp.float32),
            pltpu.VMEM((CHUNK, D), jnp.float32),
            pltpu.VMEM_SHARED((NPAD, D), jnp.float32),
            pltpu.SemaphoreType.DMA,
            pltpu.SemaphoreType.DMA,
        ],
    )
    def agg_k(g_hbm, src_hbm, dst_hbm, z_hbm, out_hbm,
              sidx, didx, rows0, rows1, agg_sp, sem0, sem1):
        c = lax.axis_index("c")
        s = lax.axis_index("s")
        wid = c * NS + s
        pltpu.sync_copy(z_hbm, agg_sp.at[pl.ds(s * RPT, RPT)])
        plsc.subcore_barrier()

        for blk in range(nblk):
            pltpu.sync_copy(src_hbm.at[wid, pl.ds(blk * IB, IB)], sidx)
            pltpu.sync_copy(dst_hbm.at[wid, pl.ds(blk * IB, IB)], didx)
            pltpu.async_copy(g_hbm.at[sidx.at[0]], rows0, sem0)

            def body(i, carry):
                j = 2 * i
                pltpu.async_copy(g_hbm.at[sidx.at[j + 1]], rows1, sem1)
                pltpu.make_async_copy(g_hbm.at[sidx.at[j]], rows0, sem0).wait()
                pltpu.sync_copy(rows0, agg_sp.at[didx.at[j]], add=True)

                @pl.when(j + 2 < IB)
                def _():
                    pltpu.async_copy(g_hbm.at[sidx.at[j + 2]], rows0, sem0)

                pltpu.make_async_copy(g_hbm.at[sidx.at[j + 1]], rows1, sem1).wait()
                pltpu.sync_copy(rows1, agg_sp.at[didx.at[j + 1]], add=True)
                return carry

            lax.fori_loop(0, IB // 2, body, 0)
        plsc.subcore_barrier()
        pltpu.sync_copy(agg_sp.at[pl.ds(s * RPT, RPT)],
                        out_hbm.at[c, pl.ds(s * RPT, RPT)])

    return agg_k(g, src3, dst3, zrows)


def _finish_call(S, g, dinv, x, b, gamma, beta):
    def fin(s_ref, g_ref, dinv_ref, x_ref, b_ref, gm_ref, bt_ref, o_ref):
        agg = (s_ref[0, :N] + s_ref[1, :N] + g_ref[...]) * dinv_ref[...] + b_ref[...]
        mean = jnp.mean(agg, axis=0, keepdims=True)
        ctr = agg - mean
        var = jnp.mean(ctr * ctr, axis=0, keepdims=True)
        xhat = ctr * lax.rsqrt(var + 1e-5)
        o_ref[...] = jnp.maximum(gm_ref[...] * xhat + bt_ref[...], 0.0) + x_ref[...]

    return pl.pallas_call(
        fin,
        out_shape=jax.ShapeDtypeStruct((N, D), jnp.float32),
    )(S, g, dinv, x, b.reshape(1, D), gamma.reshape(1, D), beta.reshape(1, D))


def kernel(x, edge_index, W, b, gamma, beta):
    E = edge_index.shape[1]
    per_w = E // NW
    nchunk = per_w // CHUNK
    assert per_w * NW == E and nchunk * CHUNK == per_w
    ei = edge_index.astype(jnp.int32)
    src3 = ei[0].reshape(NW, nchunk, CHUNK)
    dst3 = ei[1].reshape(NW, nchunk, CHUNK)
    ones_pat = jnp.zeros((CHUNK, D), jnp.float32).at[:, 0].set(1.0)
    zrows = jnp.zeros((RPT, D), jnp.float32)

    degp = _hist_call(dst3, ones_pat, zrows)
    g, dinv = _prep_call(x, W, degp)
    S = _aggregate_call(g, src3, dst3, zrows)
    return _finish_call(S, g, dinv, x, b, gamma, beta)
